# baseline (speedup 1.0000x reference)
import functools

import jax
import jax.numpy as jnp
from jax import lax
from jax.experimental import pallas as pl
from jax.experimental.pallas import tpu as pltpu
from jax.experimental.pallas import tpu_sc as plsc

N = 10000
D = 100
DP = 128
R = 16
B = 4
E = 160000
K = 128
NW = 32
CPW = 40
EPAD = NW * CPW * K
ZB = 208


def _zbuild_body(x_ref, basis_ref, att_ref, z_ref):
    r = pl.program_id(0)
    att_r = att_ref[pl.ds(r, 1), :][0]
    w = att_r[0] * basis_ref[0]
    for b in range(1, B):
        w = w + att_r[b] * basis_ref[b]
    wp = jnp.concatenate([w, jnp.zeros((D, DP - D), jnp.float32)], axis=1)
    z_ref[0] = jnp.dot(x_ref[...], wp, preferred_element_type=jnp.float32)


_BN = 2000


def _zbuild(x, basis, att):
    return pl.pallas_call(
        _zbuild_body,
        grid=(R, N // _BN),
        in_specs=[
            pl.BlockSpec((_BN, D), lambda r, n: (n, 0)),
            pl.BlockSpec((B, D, D), lambda r, n: (0, 0, 0)),
            pl.BlockSpec((R, B), lambda r, n: (0, 0)),
        ],
        out_specs=pl.BlockSpec((1, _BN, DP), lambda r, n: (r, n, 0)),
        out_shape=jax.ShapeDtypeStruct((R, N, DP), jnp.float32),
    )(x, basis, att)


def _finish_body(acc_ref, x_ref, root_ref, bias_ref, o_ref, *, relu):
    agg = acc_ref[0, :, :D] + acc_ref[1, :, :D]
    y = agg + jnp.dot(x_ref[...], root_ref[...],
                      preferred_element_type=jnp.float32) + bias_ref[0]
    if relu:
        y = jnp.maximum(y, 0.0)
    o_ref[...] = y


def _finish(acc, x, root, bias2d, relu):
    return pl.pallas_call(
        functools.partial(_finish_body, relu=relu),
        grid=(N // _BN,),
        in_specs=[
            pl.BlockSpec((2, _BN, DP), lambda n: (0, n, 0)),
            pl.BlockSpec((_BN, D), lambda n: (n, 0)),
            pl.BlockSpec((D, D), lambda n: (0, 0)),
            pl.BlockSpec((1, D), lambda n: (0, 0)),
        ],
        out_specs=pl.BlockSpec((_BN, D), lambda n: (n, 0)),
        out_shape=jax.ShapeDtypeStruct((N, D), jnp.float32),
    )(acc, x, root, bias2d)


def _sc_agg_body(z_hbm, src_hbm, dst_hbm, typ_hbm, norm_hbm, out_hbm,
                 acc_sh, rows_a, rows_b,
                 gidx_a, typ_a, dst_a, norm_a,
                 gidx_b, typ_b, dst_b, norm_b,
                 zero_v, sem_a, sem_b):
    c = lax.axis_index("c")
    s = lax.axis_index("s")
    w = c * 16 + s

    def _zrow(i, carry):
        for j in range(DP // 16):
            zero_v[i, pl.ds(j * 16, 16)] = jnp.zeros((16,), jnp.float32)
        return carry
    lax.fori_loop(0, 104, _zrow, 0)
    start = jnp.where(s < 2, s * 632, 1264 + (s - 2) * 624)
    for piece in range(6):
        pltpu.sync_copy(zero_v, acc_sh.at[pl.ds(start + piece * 104, 104)])

    @pl.when(s < 2)
    def _zero_tail():
        pltpu.sync_copy(zero_v.at[pl.ds(0, 8)], acc_sh.at[pl.ds(start + 624, 8)])
    plsc.subcore_barrier()

    def _eload(t, gidx1, typ1, dst1, norm1):
        eoff = (w + t * NW) * K
        pltpu.sync_copy(src_hbm.at[pl.ds(eoff, K)], gidx1)
        pltpu.sync_copy(typ_hbm.at[pl.ds(eoff, K)], typ1)
        pltpu.sync_copy(dst_hbm.at[pl.ds(eoff, K)], dst1)
        pltpu.sync_copy(norm_hbm.at[pl.ds(eoff, K)], norm1)
        for j in range(K // 16):
            sl = pl.ds(j * 16, 16)
            gidx1[sl] = typ1[sl] * N + gidx1[sl]

    def _issue(rows_v, gidx1, sem):
        return pltpu.async_copy(z_hbm.at[gidx1], rows_v, sem)

    def _work(rows_v, dst1, norm1):
        def _grp(g, ecarry):
            base = g * 16
            nv = norm1[pl.ds(base, 16)]
            for lane in range(16):
                nk = nv[lane]
                for j in range(DP // 16):
                    sl = pl.ds(j * 16, 16)
                    rows_v[base + lane, sl] = rows_v[base + lane, sl] * nk
            return ecarry
        lax.fori_loop(0, K // 16, _grp, 0)
        pltpu.sync_copy(rows_v, acc_sh.at[dst1], add=True)

    def _pipe(t, carry):
        _eload(t, gidx_a, typ_a, dst_a, norm_a)
        _issue(rows_a, gidx_a, sem_a).wait()
        _work(rows_a, dst_a, norm_a)
        return carry
    lax.fori_loop(0, CPW, _pipe, 0)
    plsc.subcore_barrier()
    for piece in range(3):
        pltpu.sync_copy(acc_sh.at[pl.ds(start + piece * ZB, ZB)],
                        out_hbm.at[c, pl.ds(start + piece * ZB, ZB)])

    @pl.when(s < 2)
    def _out_tail():
        pltpu.sync_copy(acc_sh.at[pl.ds(start + 624, 8)],
                        out_hbm.at[c, pl.ds(start + 624, 8)])


_sc_agg = functools.partial(
    pl.kernel,
    out_type=jax.ShapeDtypeStruct((2, N, DP), jnp.float32),
    mesh=plsc.VectorSubcoreMesh(core_axis_name="c", subcore_axis_name="s"),
    scratch_types=[
        pltpu.VMEM_SHARED((N, DP), jnp.float32),
        pltpu.VMEM((K, DP), jnp.float32),
        pltpu.VMEM((K, DP), jnp.float32),
        pltpu.VMEM((K,), jnp.int32),
        pltpu.VMEM((K,), jnp.int32),
        pltpu.VMEM((K,), jnp.int32),
        pltpu.VMEM((K,), jnp.float32),
        pltpu.VMEM((K,), jnp.int32),
        pltpu.VMEM((K,), jnp.int32),
        pltpu.VMEM((K,), jnp.int32),
        pltpu.VMEM((K,), jnp.float32),
        pltpu.VMEM((104, DP), jnp.float32),
        pltpu.SemaphoreType.DMA,
        pltpu.SemaphoreType.DMA,
    ],
)(_sc_agg_body)


def kernel(entity, edge_idx, edge_type, edge_norm, emb,
           basis1, att1, root1, bias1, basis2, att2, root2, bias2):
    x = jnp.take(emb, entity, axis=0)
    pad = EPAD - E
    src = jnp.pad(edge_idx[0], (0, pad))
    dst = jnp.concatenate(
        [edge_idx[1], jnp.arange(pad, dtype=jnp.int32) % jnp.int32(N)])
    typ = jnp.pad(edge_type, (0, pad))
    norm = jnp.pad(edge_norm, (0, pad))

    def layer(x, basis, att, root, bias, relu):
        z = _zbuild(x, basis, att).reshape(R * N, DP)
        acc = _sc_agg(z, src, dst, typ, norm)
        return _finish(acc, x, root, bias.reshape(1, D), relu)

    x = layer(x, basis1, att1, root1, bias1, False)
    x = layer(x, basis1, att1, root1, bias1, True)
    return layer(x, basis2, att2, root2, bias2, False)


# speedup vs baseline: 1.4405x; 1.4405x over previous
import functools

import jax
import jax.numpy as jnp
from jax import lax
from jax.experimental import pallas as pl
from jax.experimental.pallas import tpu as pltpu
from jax.experimental.pallas import tpu_sc as plsc

N = 10000
D = 100
DP = 128
R = 16
B = 4
E = 160000
K = 128
NW = 32
BASE_CHUNKS = (E // K) // NW
EXTRA_W = (E // K) - BASE_CHUNKS * NW
ZB = 208


def _zbuild_body(x_ref, basis_ref, att_ref, z_ref):
    r = pl.program_id(0)
    att_r = att_ref[pl.ds(r, 1), :][0]
    w = att_r[0] * basis_ref[0]
    for b in range(1, B):
        w = w + att_r[b] * basis_ref[b]
    wp = jnp.concatenate([w, jnp.zeros((D, DP - D), jnp.float32)], axis=1)
    z_ref[0] = jnp.dot(x_ref[...], wp, preferred_element_type=jnp.float32)


_BN = 2000


def _zbuild(x, basis, att):
    return pl.pallas_call(
        _zbuild_body,
        grid=(R, N // _BN),
        in_specs=[
            pl.BlockSpec((_BN, D), lambda r, n: (n, 0)),
            pl.BlockSpec((B, D, D), lambda r, n: (0, 0, 0)),
            pl.BlockSpec((R, B), lambda r, n: (0, 0)),
        ],
        out_specs=pl.BlockSpec((1, _BN, DP), lambda r, n: (r, n, 0)),
        out_shape=jax.ShapeDtypeStruct((R, N, DP), jnp.float32),
    )(x, basis, att)


def _finish_body(acc_ref, x_ref, root_ref, bias_ref, o_ref, *, relu):
    agg = acc_ref[0, :, :D] + acc_ref[1, :, :D]
    y = agg + jnp.dot(x_ref[...], root_ref[...],
                      preferred_element_type=jnp.float32) + bias_ref[0]
    if relu:
        y = jnp.maximum(y, 0.0)
    o_ref[...] = y


def _finish(acc, x, root, bias2d, relu):
    return pl.pallas_call(
        functools.partial(_finish_body, relu=relu),
        grid=(N // _BN,),
        in_specs=[
            pl.BlockSpec((2, _BN, DP), lambda n: (0, n, 0)),
            pl.BlockSpec((_BN, D), lambda n: (n, 0)),
            pl.BlockSpec((D, D), lambda n: (0, 0)),
            pl.BlockSpec((1, D), lambda n: (0, 0)),
        ],
        out_specs=pl.BlockSpec((_BN, D), lambda n: (n, 0)),
        out_shape=jax.ShapeDtypeStruct((N, D), jnp.float32),
    )(acc, x, root, bias2d)


def _sc_agg_body(z_hbm, src_hbm, dst_hbm, typ_hbm, norm_hbm, out_hbm,
                 acc_sh, rows_v, gidx_v, aux_v, dst_v, norm_v, zero_v, sem):
    c = lax.axis_index("c")
    s = lax.axis_index("s")
    w = c * 16 + s

    def _zrow(i, carry):
        for j in range(DP // 16):
            zero_v[i, pl.ds(j * 16, 16)] = jnp.zeros((16,), jnp.float32)
        return carry
    lax.fori_loop(0, ZB, _zrow, 0)
    start = jnp.where(s < 2, s * 632, 1264 + (s - 2) * 624)
    for piece in range(3):
        pltpu.sync_copy(zero_v, acc_sh.at[pl.ds(start + piece * ZB, ZB)])

    @pl.when(s < 2)
    def _zero_tail():
        pltpu.sync_copy(zero_v.at[pl.ds(0, 8)], acc_sh.at[pl.ds(start + 624, 8)])
    plsc.subcore_barrier()

    nchunks = BASE_CHUNKS + jnp.where(w < EXTRA_W, 1, 0)

    def _chunk(t, carry):
        eoff = (w + t * NW) * K
        pltpu.sync_copy(src_hbm.at[pl.ds(eoff, K)], gidx_v)
        pltpu.sync_copy(typ_hbm.at[pl.ds(eoff, K)], aux_v)
        pltpu.sync_copy(dst_hbm.at[pl.ds(eoff, K)], dst_v)
        pltpu.sync_copy(norm_hbm.at[pl.ds(eoff, K)], norm_v)
        for j in range(K // 16):
            sl = pl.ds(j * 16, 16)
            gidx_v[sl] = aux_v[sl] * N + gidx_v[sl]
        pltpu.async_copy(z_hbm.at[gidx_v], rows_v, sem).wait()
        def _grp(g, ecarry):
            base = g * 16
            nv = norm_v[pl.ds(base, 16)]
            for lane in range(16):
                nk = nv[lane]
                for j in range(DP // 16):
                    sl = pl.ds(j * 16, 16)
                    rows_v[base + lane, sl] = rows_v[base + lane, sl] * nk
            return ecarry
        lax.fori_loop(0, K // 16, _grp, 0)
        pltpu.sync_copy(rows_v, acc_sh.at[dst_v], add=True)
        return carry
    lax.fori_loop(0, nchunks, _chunk, 0)
    plsc.subcore_barrier()
    for piece in range(3):
        pltpu.sync_copy(acc_sh.at[pl.ds(start + piece * ZB, ZB)],
                        out_hbm.at[c, pl.ds(start + piece * ZB, ZB)])

    @pl.when(s < 2)
    def _out_tail():
        pltpu.sync_copy(acc_sh.at[pl.ds(start + 624, 8)],
                        out_hbm.at[c, pl.ds(start + 624, 8)])


_sc_agg = functools.partial(
    pl.kernel,
    out_type=jax.ShapeDtypeStruct((2, N, DP), jnp.float32),
    mesh=plsc.VectorSubcoreMesh(core_axis_name="c", subcore_axis_name="s"),
    scratch_types=[
        pltpu.VMEM_SHARED((N, DP), jnp.float32),
        pltpu.VMEM((K, DP), jnp.float32),
        pltpu.VMEM((K,), jnp.int32),
        pltpu.VMEM((K,), jnp.int32),
        pltpu.VMEM((K,), jnp.int32),
        pltpu.VMEM((K,), jnp.float32),
        pltpu.VMEM((ZB, DP), jnp.float32),
        pltpu.SemaphoreType.DMA,
    ],
)(_sc_agg_body)


def kernel(entity, edge_idx, edge_type, edge_norm, emb,
           basis1, att1, root1, bias1, basis2, att2, root2, bias2):
    x = jnp.take(emb, entity, axis=0)
    src = edge_idx[0]
    dst = edge_idx[1]
    typ = edge_type
    norm = edge_norm

    def layer(x, basis, att, root, bias, relu):
        z = _zbuild(x, basis, att).reshape(R * N, DP)
        acc = _sc_agg(z, src, dst, typ, norm)
        return _finish(acc, x, root, bias.reshape(1, D), relu)

    x = layer(x, basis1, att1, root1, bias1, False)
    x = layer(x, basis1, att1, root1, bias1, True)
    return layer(x, basis2, att2, root2, bias2, False)


# speedup vs baseline: 1.6912x; 1.1741x over previous
import functools

import jax
import jax.numpy as jnp
from jax import lax
from jax.experimental import pallas as pl
from jax.experimental.pallas import tpu as pltpu
from jax.experimental.pallas import tpu_sc as plsc

N = 10000
D = 100
DP = 128
R = 16
B = 4
E = 160000
K = 128
NW = 32
CPW = 40
EPAD = NW * CPW * K
ZB = 208


def _zbuild_body(x_ref, basis_ref, att_ref, z_ref):
    r = pl.program_id(0)
    att_r = att_ref[pl.ds(r, 1), :][0]
    w = att_r[0] * basis_ref[0]
    for b in range(1, B):
        w = w + att_r[b] * basis_ref[b]
    wp = jnp.concatenate([w, jnp.zeros((D, DP - D), jnp.float32)], axis=1)
    z_ref[0] = jnp.dot(x_ref[...], wp, preferred_element_type=jnp.float32)


_BN = 2000


def _zbuild(x, basis, att):
    return pl.pallas_call(
        _zbuild_body,
        grid=(R, N // _BN),
        in_specs=[
            pl.BlockSpec((_BN, D), lambda r, n: (n, 0)),
            pl.BlockSpec((B, D, D), lambda r, n: (0, 0, 0)),
            pl.BlockSpec((R, B), lambda r, n: (0, 0)),
        ],
        out_specs=pl.BlockSpec((1, _BN, DP), lambda r, n: (r, n, 0)),
        out_shape=jax.ShapeDtypeStruct((R, N, DP), jnp.float32),
    )(x, basis, att)


def _finish_body(acc_ref, x_ref, root_ref, bias_ref, o_ref, *, relu):
    agg = acc_ref[0, :, :D] + acc_ref[1, :, :D]
    y = agg + jnp.dot(x_ref[...], root_ref[...],
                      preferred_element_type=jnp.float32) + bias_ref[0]
    if relu:
        y = jnp.maximum(y, 0.0)
    o_ref[...] = y


def _finish(acc, x, root, bias2d, relu):
    return pl.pallas_call(
        functools.partial(_finish_body, relu=relu),
        grid=(N // _BN,),
        in_specs=[
            pl.BlockSpec((2, _BN, DP), lambda n: (0, n, 0)),
            pl.BlockSpec((_BN, D), lambda n: (n, 0)),
            pl.BlockSpec((D, D), lambda n: (0, 0)),
            pl.BlockSpec((1, D), lambda n: (0, 0)),
        ],
        out_specs=pl.BlockSpec((_BN, D), lambda n: (n, 0)),
        out_shape=jax.ShapeDtypeStruct((N, D), jnp.float32),
    )(acc, x, root, bias2d)


def _sc_agg_body(z_hbm, src_hbm, dst_hbm, typ_hbm, norm_hbm, out_hbm,
                 acc_sh, rows_a, rows_b,
                 gidx_a, typ_a, dst_a, norm_a,
                 gidx_b, typ_b, dst_b, norm_b,
                 zero_v, sem_a, sem_b):
    c = lax.axis_index("c")
    s = lax.axis_index("s")
    w = c * 16 + s

    def _zrow(i, carry):
        for j in range(DP // 16):
            zero_v[i, pl.ds(j * 16, 16)] = jnp.zeros((16,), jnp.float32)
        return carry
    lax.fori_loop(0, 104, _zrow, 0)
    start = jnp.where(s < 2, s * 632, 1264 + (s - 2) * 624)
    for piece in range(6):
        pltpu.sync_copy(zero_v, acc_sh.at[pl.ds(start + piece * 104, 104)])

    @pl.when(s < 2)
    def _zero_tail():
        pltpu.sync_copy(zero_v.at[pl.ds(0, 8)], acc_sh.at[pl.ds(start + 624, 8)])
    plsc.subcore_barrier()

    def _eload(t, gidx1, typ1, dst1, norm1):
        eoff = (w + t * NW) * K
        pltpu.sync_copy(src_hbm.at[pl.ds(eoff, K)], gidx1)
        pltpu.sync_copy(typ_hbm.at[pl.ds(eoff, K)], typ1)
        pltpu.sync_copy(dst_hbm.at[pl.ds(eoff, K)], dst1)
        pltpu.sync_copy(norm_hbm.at[pl.ds(eoff, K)], norm1)
        for j in range(K // 16):
            sl = pl.ds(j * 16, 16)
            gidx1[sl] = typ1[sl] * N + gidx1[sl]

    def _issue(rows_v, gidx1, sem):
        return pltpu.async_copy(z_hbm.at[gidx1], rows_v, sem)

    def _work(rows_v, dst1, norm1):
        def _grp(g, ecarry):
            base = g * 16
            nv = norm1[pl.ds(base, 16)]
            for lane in range(16):
                nk = nv[lane]
                for j in range(DP // 16):
                    sl = pl.ds(j * 16, 16)
                    rows_v[base + lane, sl] = rows_v[base + lane, sl] * nk
            return ecarry
        lax.fori_loop(0, K // 16, _grp, 0)
        pltpu.sync_copy(rows_v, acc_sh.at[dst1], add=True)

    def _pipe(i, carry):
        t0 = 4 * i
        _eload(t0, gidx_a, typ_a, dst_a, norm_a)
        cp_a = _issue(rows_a, gidx_a, sem_a)
        _eload(t0 + 1, gidx_b, typ_b, dst_b, norm_b)
        cp_b = _issue(rows_b, gidx_b, sem_b)
        cp_a.wait()
        _work(rows_a, dst_a, norm_a)
        _eload(t0 + 2, gidx_a, typ_a, dst_a, norm_a)
        cp_a = _issue(rows_a, gidx_a, sem_a)
        cp_b.wait()
        _work(rows_b, dst_b, norm_b)
        _eload(t0 + 3, gidx_b, typ_b, dst_b, norm_b)
        cp_b = _issue(rows_b, gidx_b, sem_b)
        cp_a.wait()
        _work(rows_a, dst_a, norm_a)
        cp_b.wait()
        _work(rows_b, dst_b, norm_b)
        return carry
    lax.fori_loop(0, CPW // 4, _pipe, 0)
    plsc.subcore_barrier()
    for piece in range(3):
        pltpu.sync_copy(acc_sh.at[pl.ds(start + piece * ZB, ZB)],
                        out_hbm.at[c, pl.ds(start + piece * ZB, ZB)])

    @pl.when(s < 2)
    def _out_tail():
        pltpu.sync_copy(acc_sh.at[pl.ds(start + 624, 8)],
                        out_hbm.at[c, pl.ds(start + 624, 8)])


_sc_agg = functools.partial(
    pl.kernel,
    out_type=jax.ShapeDtypeStruct((2, N, DP), jnp.float32),
    mesh=plsc.VectorSubcoreMesh(core_axis_name="c", subcore_axis_name="s"),
    scratch_types=[
        pltpu.VMEM_SHARED((N, DP), jnp.float32),
        pltpu.VMEM((K, DP), jnp.float32),
        pltpu.VMEM((K, DP), jnp.float32),
        pltpu.VMEM((K,), jnp.int32),
        pltpu.VMEM((K,), jnp.int32),
        pltpu.VMEM((K,), jnp.int32),
        pltpu.VMEM((K,), jnp.float32),
        pltpu.VMEM((K,), jnp.int32),
        pltpu.VMEM((K,), jnp.int32),
        pltpu.VMEM((K,), jnp.int32),
        pltpu.VMEM((K,), jnp.float32),
        pltpu.VMEM((104, DP), jnp.float32),
        pltpu.SemaphoreType.DMA,
        pltpu.SemaphoreType.DMA,
    ],
)(_sc_agg_body)


def kernel(entity, edge_idx, edge_type, edge_norm, emb,
           basis1, att1, root1, bias1, basis2, att2, root2, bias2):
    x = jnp.take(emb, entity, axis=0)
    ar = jnp.arange(EPAD - E, dtype=jnp.int32)
    src = jnp.concatenate([edge_idx[0], ar % jnp.int32(N)])
    dst = jnp.concatenate([edge_idx[1], (ar * 7) % jnp.int32(N)])
    typ = jnp.concatenate([edge_type, ar % jnp.int32(R)])
    norm = jnp.pad(edge_norm, (0, EPAD - E))

    def layer(x, basis, att, root, bias, relu):
        z = _zbuild(x, basis, att).reshape(R * N, DP)
        acc = _sc_agg(z, src, dst, typ, norm)
        return _finish(acc, x, root, bias.reshape(1, D), relu)

    x = layer(x, basis1, att1, root1, bias1, False)
    x = layer(x, basis1, att1, root1, bias1, True)
    return layer(x, basis2, att2, root2, bias2, False)
